# Initial kernel scaffold; baseline (speedup 1.0000x reference)
import jax
import jax.numpy as jnp
from jax.experimental import pallas as pl


def kernel(x, edge_index, W1, b1, W2, b2):
    raise NotImplementedError("write your pallas kernel here")


# speedup vs baseline: 19.7331x; 19.7331x over previous
import functools

import jax
import jax.numpy as jnp
from jax import lax
from jax.experimental import pallas as pl
from jax.experimental.pallas import tpu as pltpu
from jax.experimental.pallas import tpu_sc as plsc

N = 10000
E = 320000
F_IN = 128
HID = 16
CLS = 40
CLSP = 48

NC, NS, L = 2, 16, 16
NW = NC * NS
CHUNK = 128
NCHUNK = E // CHUNK
ROUNDS = (NCHUNK + NW - 1) // NW
NPAD = 10240
RPT = NPAD // NS


def _sc_mesh():
    return plsc.VectorSubcoreMesh(
        core_axis_name="c", subcore_axis_name="s", num_cores=NC, num_subcores=NS
    )


_SC_PARAMS = pltpu.CompilerParams(use_tc_tiling_on_sc=False)


def _make_propagate(F):
    def body(feat_hbm, src_hbm, dst_hbm, z_hbm, out_hbm, acc, src_v, dst_v, rows_v):
        c = lax.axis_index("c")
        s = lax.axis_index("s")
        wid = s * NC + c
        sl = pl.ds(s * RPT, RPT)
        pltpu.sync_copy(z_hbm.at[sl], acc.at[sl])
        plsc.subcore_barrier()

        def round_body(j, carry):
            cidx = j * NW + wid

            @pl.when(cidx < NCHUNK)
            def _():
                pltpu.sync_copy(src_hbm.at[cidx], src_v)
                pltpu.sync_copy(dst_hbm.at[cidx], dst_v)
                pltpu.sync_copy(feat_hbm.at[src_v], rows_v)
                pltpu.sync_copy(rows_v, acc.at[dst_v], add=True)

            return carry

        lax.fori_loop(0, ROUNDS, round_body, 0)
        plsc.subcore_barrier()
        pltpu.sync_copy(acc.at[sl], out_hbm.at[c, sl])

    return pl.kernel(
        body,
        out_type=jax.ShapeDtypeStruct((NC, NPAD, F), jnp.float32),
        mesh=_sc_mesh(),
        compiler_params=_SC_PARAMS,
        scratch_types=[
            pltpu.VMEM_SHARED((NPAD, F), jnp.float32),
            pltpu.VMEM((CHUNK,), jnp.int32),
            pltpu.VMEM((CHUNK,), jnp.int32),
            pltpu.VMEM((CHUNK, F), jnp.float32),
        ],
    )


def _make_degree():
    def body(dst_hbm, z_hbm, out_hbm, acc, dst_v, rows_v):
        c = lax.axis_index("c")
        s = lax.axis_index("s")
        wid = s * NC + c
        sl = pl.ds(s * RPT, RPT)
        pltpu.sync_copy(z_hbm.at[sl], acc.at[sl])

        def ones_body(i, carry):
            rows_v[i, :] = jnp.ones((L,), jnp.float32)
            return carry

        lax.fori_loop(0, CHUNK, ones_body, 0)
        plsc.subcore_barrier()

        def round_body(j, carry):
            cidx = j * NW + wid

            @pl.when(cidx < NCHUNK)
            def _():
                pltpu.sync_copy(dst_hbm.at[cidx], dst_v)
                pltpu.sync_copy(rows_v, acc.at[dst_v], add=True)

            return carry

        lax.fori_loop(0, ROUNDS, round_body, 0)
        plsc.subcore_barrier()
        pltpu.sync_copy(acc.at[sl], out_hbm.at[c, sl])

    return pl.kernel(
        body,
        out_type=jax.ShapeDtypeStruct((NC, NPAD, L), jnp.float32),
        mesh=_sc_mesh(),
        compiler_params=_SC_PARAMS,
        scratch_types=[
            pltpu.VMEM_SHARED((NPAD, L), jnp.float32),
            pltpu.VMEM((CHUNK,), jnp.int32),
            pltpu.VMEM((CHUNK, L), jnp.float32),
        ],
    )


_propagate16 = _make_propagate(HID)
_propagate48 = _make_propagate(CLSP)
_degree = _make_degree()

_BM = 1000
_GRID = N // _BM


def _stage1(x, w1, degp):
    def body(x_ref, w1_ref, degp_ref, hs_ref, dinv_ref):
        deg = degp_ref[0, :, 0:1] + degp_ref[1, :, 0:1] + 1.0
        dinv = lax.rsqrt(deg)
        h = jnp.dot(x_ref[...], w1_ref[...], preferred_element_type=jnp.float32)
        hs_ref[...] = h * dinv
        dinv_ref[...] = dinv

    return pl.pallas_call(
        body,
        grid=(_GRID,),
        in_specs=[
            pl.BlockSpec((_BM, F_IN), lambda i: (i, 0)),
            pl.BlockSpec((F_IN, HID), lambda i: (0, 0)),
            pl.BlockSpec((NC, _BM, L), lambda i: (0, i, 0)),
        ],
        out_specs=[
            pl.BlockSpec((_BM, HID), lambda i: (i, 0)),
            pl.BlockSpec((_BM, 1), lambda i: (i, 0)),
        ],
        out_shape=[
            jax.ShapeDtypeStruct((N, HID), jnp.float32),
            jax.ShapeDtypeStruct((N, 1), jnp.float32),
        ],
    )(x, w1, degp)


def _stage2(p1, hs, dinv, b1r, w2p):
    def body(p_ref, hs_ref, dinv_ref, b1_ref, w2_ref, hs2_ref):
        dv = dinv_ref[...]
        sacc = dv * (p_ref[0] + p_ref[1] + hs_ref[...]) + b1_ref[...]
        h1 = jnp.maximum(sacc, 0.0)
        hs2_ref[...] = (
            jnp.dot(h1, w2_ref[...], preferred_element_type=jnp.float32) * dv
        )

    return pl.pallas_call(
        body,
        grid=(_GRID,),
        in_specs=[
            pl.BlockSpec((NC, _BM, HID), lambda i: (0, i, 0)),
            pl.BlockSpec((_BM, HID), lambda i: (i, 0)),
            pl.BlockSpec((_BM, 1), lambda i: (i, 0)),
            pl.BlockSpec((1, HID), lambda i: (0, 0)),
            pl.BlockSpec((HID, CLSP), lambda i: (0, 0)),
        ],
        out_specs=pl.BlockSpec((_BM, CLSP), lambda i: (i, 0)),
        out_shape=jax.ShapeDtypeStruct((N, CLSP), jnp.float32),
    )(p1, hs, dinv, b1r, w2p)


def _stage3(p2, hs2, dinv, b2r):
    def body(p_ref, hs2_ref, dinv_ref, b2_ref, out_ref, logit_ref):
        t = (dinv_ref[...] * (p_ref[0] + p_ref[1] + hs2_ref[...]))[:, :CLS]
        t = t + b2_ref[...]
        m = jnp.max(t, axis=1, keepdims=True)
        e = jnp.exp(t - m)
        out_ref[...] = e / jnp.sum(e, axis=1, keepdims=True)
        logit_ref[...] = t

    return pl.pallas_call(
        body,
        grid=(_GRID,),
        in_specs=[
            pl.BlockSpec((NC, _BM, CLSP), lambda i: (0, i, 0)),
            pl.BlockSpec((_BM, CLSP), lambda i: (i, 0)),
            pl.BlockSpec((_BM, 1), lambda i: (i, 0)),
            pl.BlockSpec((1, CLS), lambda i: (0, 0)),
        ],
        out_specs=[
            pl.BlockSpec((_BM, CLS), lambda i: (i, 0)),
            pl.BlockSpec((_BM, CLS), lambda i: (i, 0)),
        ],
        out_shape=[
            jax.ShapeDtypeStruct((N, CLS), jnp.float32),
            jax.ShapeDtypeStruct((N, CLS), jnp.float32),
        ],
    )(p2, hs2, dinv, b2r)


def kernel(x, edge_index, W1, b1, W2, b2):
    src2d = edge_index[0].reshape(NCHUNK, CHUNK)
    dst2d = edge_index[1].reshape(NCHUNK, CHUNK)
    z16 = jnp.zeros((NPAD, HID), jnp.float32)
    z48 = jnp.zeros((NPAD, CLSP), jnp.float32)
    w2p = jnp.zeros((HID, CLSP), jnp.float32).at[:, :CLS].set(W2)
    b1r = b1.reshape(1, HID)
    b2r = b2.reshape(1, CLS)

    degp = _degree(dst2d, z16)
    hs, dinv = _stage1(x, W1, degp)
    p1 = _propagate16(hs, src2d, dst2d, z16)
    hs2 = _stage2(p1, hs, dinv, b1r, w2p)
    p2 = _propagate48(hs2, src2d, dst2d, z48)
    out, logits = _stage3(p2, hs2, dinv, b2r)
    return (out, logits)


# speedup vs baseline: 45.3591x; 2.2986x over previous
import jax
import jax.numpy as jnp
from jax import lax
from jax.experimental import pallas as pl
from jax.experimental.pallas import tpu as pltpu
from jax.experimental.pallas import tpu_sc as plsc

N = 10000
E = 320000
F_IN = 128
HID = 16
CLS = 40
CLSP = 48

NC, NS, L = 2, 16, 16
NW = NC * NS
CHUNK = 128
CPW = 80
EPAD = NW * CPW * CHUNK
NBUF = 4
NG = CPW // NBUF
NG2 = NG // 2
NPAD = 10240
RPT = NPAD // NS


def _sc_mesh():
    return plsc.VectorSubcoreMesh(
        core_axis_name="c", subcore_axis_name="s", num_cores=NC, num_subcores=NS
    )


_SC_PARAMS = pltpu.CompilerParams(use_tc_tiling_on_sc=False)


def _make_propagate(F):
    def body(feat_hbm, src_hbm, dst_hbm, z_hbm, out_hbm, acc, src_v, dst_v, rows, gsem, ssem):
        c = lax.axis_index("c")
        s = lax.axis_index("s")
        wid = s * NC + c
        sl = pl.ds(s * RPT, RPT)
        pltpu.sync_copy(z_hbm.at[sl], acc.at[sl])
        pltpu.sync_copy(src_hbm.at[wid], src_v)
        pltpu.sync_copy(dst_hbm.at[wid], dst_v)
        plsc.subcore_barrier()

        def super_body(u, carry):
            base = u * 2 * NBUF
            gds = [
                pltpu.async_copy(
                    feat_hbm.at[src_v.at[base + i]], rows[i // NBUF][i % NBUF],
                    gsem[i // NBUF][i % NBUF],
                )
                for i in range(2 * NBUF)
            ]
            sds = []
            for i in range(2 * NBUF):
                gds[i].wait()
                sds.append(
                    pltpu.async_copy(
                        rows[i // NBUF][i % NBUF], acc.at[dst_v.at[base + i]],
                        ssem[i // NBUF][i % NBUF], add=True,
                    )
                )
            for d in sds:
                d.wait()
            return carry

        lax.fori_loop(0, NG2, super_body, 0)
        plsc.subcore_barrier()
        pltpu.sync_copy(acc.at[sl], out_hbm.at[c, sl])

    return pl.kernel(
        body,
        out_type=jax.ShapeDtypeStruct((NC, NPAD, F), jnp.float32),
        mesh=_sc_mesh(),
        compiler_params=_SC_PARAMS,
        scratch_types=[
            pltpu.VMEM_SHARED((NPAD, F), jnp.float32),
            pltpu.VMEM((CPW, CHUNK), jnp.int32),
            pltpu.VMEM((CPW, CHUNK), jnp.int32),
            [[pltpu.VMEM((CHUNK, F), jnp.float32) for _ in range(NBUF)] for _ in range(2)],
            [[pltpu.SemaphoreType.DMA for _ in range(NBUF)] for _ in range(2)],
            [[pltpu.SemaphoreType.DMA for _ in range(NBUF)] for _ in range(2)],
        ],
    )


def _make_degree():
    def body(dst_hbm, z_hbm, out_hbm, acc, dst_v, ones_v):
        c = lax.axis_index("c")
        s = lax.axis_index("s")
        wid = s * NC + c
        sl = pl.ds(s * RPT, RPT)
        pltpu.sync_copy(z_hbm.at[sl], acc.at[sl])
        pltpu.sync_copy(dst_hbm.at[wid], dst_v)

        def ones_body(i, carry):
            ones_v[i, :] = jnp.ones((L,), jnp.float32)
            return carry

        lax.fori_loop(0, CHUNK, ones_body, 0)
        plsc.subcore_barrier()

        def jbody(j, carry):
            pltpu.sync_copy(ones_v, acc.at[dst_v.at[j]], add=True)
            return carry

        lax.fori_loop(0, CPW, jbody, 0)
        plsc.subcore_barrier()
        pltpu.sync_copy(acc.at[sl], out_hbm.at[c, sl])

    return pl.kernel(
        body,
        out_type=jax.ShapeDtypeStruct((NC, NPAD, L), jnp.float32),
        mesh=_sc_mesh(),
        compiler_params=_SC_PARAMS,
        scratch_types=[
            pltpu.VMEM_SHARED((NPAD, L), jnp.float32),
            pltpu.VMEM((CPW, CHUNK), jnp.int32),
            pltpu.VMEM((CHUNK, L), jnp.float32),
        ],
    )


_propagate16 = _make_propagate(HID)
_propagate48 = _make_propagate(CLSP)
_degree = _make_degree()

_BM = 1000
_GRID = N // _BM


def _stage1(x, w1, degp):
    def body(x_ref, w1_ref, degp_ref, hs_ref, dinv_ref):
        deg = degp_ref[0, :, 0:1] + degp_ref[1, :, 0:1] + 1.0
        dinv = lax.rsqrt(deg)
        h = jnp.dot(x_ref[...], w1_ref[...], preferred_element_type=jnp.float32)
        hs_ref[...] = h * dinv
        dinv_ref[...] = dinv

    return pl.pallas_call(
        body,
        grid=(_GRID,),
        in_specs=[
            pl.BlockSpec((_BM, F_IN), lambda i: (i, 0)),
            pl.BlockSpec((F_IN, HID), lambda i: (0, 0)),
            pl.BlockSpec((NC, _BM, L), lambda i: (0, i, 0)),
        ],
        out_specs=[
            pl.BlockSpec((_BM, HID), lambda i: (i, 0)),
            pl.BlockSpec((_BM, 1), lambda i: (i, 0)),
        ],
        out_shape=[
            jax.ShapeDtypeStruct((N, HID), jnp.float32),
            jax.ShapeDtypeStruct((N, 1), jnp.float32),
        ],
    )(x, w1, degp)


def _stage2(p1, hs, dinv, b1r, w2p):
    def body(p_ref, hs_ref, dinv_ref, b1_ref, w2_ref, hs2_ref):
        dv = dinv_ref[...]
        sacc = dv * (p_ref[0] + p_ref[1] + hs_ref[...]) + b1_ref[...]
        h1 = jnp.maximum(sacc, 0.0)
        hs2_ref[...] = (
            jnp.dot(h1, w2_ref[...], preferred_element_type=jnp.float32) * dv
        )

    return pl.pallas_call(
        body,
        grid=(_GRID,),
        in_specs=[
            pl.BlockSpec((NC, _BM, HID), lambda i: (0, i, 0)),
            pl.BlockSpec((_BM, HID), lambda i: (i, 0)),
            pl.BlockSpec((_BM, 1), lambda i: (i, 0)),
            pl.BlockSpec((1, HID), lambda i: (0, 0)),
            pl.BlockSpec((HID, CLSP), lambda i: (0, 0)),
        ],
        out_specs=pl.BlockSpec((_BM, CLSP), lambda i: (i, 0)),
        out_shape=jax.ShapeDtypeStruct((N, CLSP), jnp.float32),
    )(p1, hs, dinv, b1r, w2p)


def _stage3(p2, hs2, dinv, b2r):
    def body(p_ref, hs2_ref, dinv_ref, b2_ref, out_ref, logit_ref):
        t = (dinv_ref[...] * (p_ref[0] + p_ref[1] + hs2_ref[...]))[:, :CLS]
        t = t + b2_ref[...]
        m = jnp.max(t, axis=1, keepdims=True)
        e = jnp.exp(t - m)
        out_ref[...] = e / jnp.sum(e, axis=1, keepdims=True)
        logit_ref[...] = t

    return pl.pallas_call(
        body,
        grid=(_GRID,),
        in_specs=[
            pl.BlockSpec((NC, _BM, CLSP), lambda i: (0, i, 0)),
            pl.BlockSpec((_BM, CLSP), lambda i: (i, 0)),
            pl.BlockSpec((_BM, 1), lambda i: (i, 0)),
            pl.BlockSpec((1, CLS), lambda i: (0, 0)),
        ],
        out_specs=[
            pl.BlockSpec((_BM, CLS), lambda i: (i, 0)),
            pl.BlockSpec((_BM, CLS), lambda i: (i, 0)),
        ],
        out_shape=[
            jax.ShapeDtypeStruct((N, CLS), jnp.float32),
            jax.ShapeDtypeStruct((N, CLS), jnp.float32),
        ],
    )(p2, hs2, dinv, b2r)


def kernel(x, edge_index, W1, b1, W2, b2):
    npad_extra = jnp.arange(EPAD - E, dtype=jnp.int32)
    src3d = jnp.concatenate(
        [edge_index[0], npad_extra % N]
    ).reshape(NW, CPW, CHUNK)
    dst3d = jnp.concatenate(
        [edge_index[1], N + npad_extra % (NPAD - N)]
    ).reshape(NW, CPW, CHUNK)
    z16 = jnp.zeros((NPAD, HID), jnp.float32)
    z48 = jnp.zeros((NPAD, CLSP), jnp.float32)
    w2p = jnp.zeros((HID, CLSP), jnp.float32).at[:, :CLS].set(W2)
    b1r = b1.reshape(1, HID)
    b2r = b2.reshape(1, CLS)

    degp = _degree(dst3d, z16)
    hs, dinv = _stage1(x, W1, degp)
    p1 = _propagate16(hs, src3d, dst3d, z16)
    hs2 = _stage2(p1, hs, dinv, b1r, w2p)
    p2 = _propagate48(hs2, src3d, dst3d, z48)
    out, logits = _stage3(p2, hs2, dinv, b2r)
    return (out, logits)


# speedup vs baseline: 52.7914x; 1.1639x over previous
import jax
import jax.numpy as jnp
from jax import lax
from jax.experimental import pallas as pl
from jax.experimental.pallas import tpu as pltpu
from jax.experimental.pallas import tpu_sc as plsc

N = 10000
E = 320000
F_IN = 128
HID = 16
CLS = 40
CLSP = 48

NC, NS, L = 2, 16, 16
NW = NC * NS
CHUNK = 128
NCHUNK = E // CHUNK
CPWB = NCHUNK // NW
NBUF = 4
NG2 = (CPWB + 1 + 2 * NBUF - 1) // (2 * NBUF)
NPAD = 10240
RPT = NPAD // NS


def _sc_mesh():
    return plsc.VectorSubcoreMesh(
        core_axis_name="c", subcore_axis_name="s", num_cores=NC, num_subcores=NS
    )


_SC_PARAMS = pltpu.CompilerParams(use_tc_tiling_on_sc=False)


def _make_propagate(F, split_out):
    nsplit = F // L if split_out else 1

    def body(feat_hbm, edge_hbm, *rest):
        outs = rest[:nsplit]
        acc, src_v, dst_v, zbuf, rows, gsem, ssem = rest[nsplit:]
        c = lax.axis_index("c")
        s = lax.axis_index("s")
        wid = s * NC + c
        cnt = jnp.where(wid < NCHUNK - NW * CPWB, CPWB + 1, CPWB)
        start = CPWB * wid + jnp.minimum(wid, NCHUNK - NW * CPWB)
        sl = pl.ds(s * RPT, RPT)

        def zfill(i, carry):
            for k in range(F // L):
                zbuf[i, pl.ds(k * L, L)] = jnp.zeros((L,), jnp.float32)
            return carry

        lax.fori_loop(0, 128, zfill, 0)
        for k in range(RPT // 128):
            pltpu.sync_copy(zbuf, acc.at[pl.ds(s * RPT + k * 128, 128)])
        pltpu.sync_copy(edge_hbm.at[0, pl.ds(start, CPWB)], src_v.at[pl.ds(0, CPWB)])
        pltpu.sync_copy(edge_hbm.at[1, pl.ds(start, CPWB)], dst_v.at[pl.ds(0, CPWB)])

        @pl.when(cnt > CPWB)
        def _():
            pltpu.sync_copy(edge_hbm.at[0, start + CPWB], src_v.at[CPWB])
            pltpu.sync_copy(edge_hbm.at[1, start + CPWB], dst_v.at[CPWB])

        plsc.subcore_barrier()

        def super_body(u, carry):
            base = u * 2 * NBUF
            gds = []
            for i in range(2 * NBUF):
                d = pltpu.async_copy(
                    feat_hbm.at[src_v.at[base + i]], rows[i // NBUF][i % NBUF],
                    gsem[i // NBUF][i % NBUF],
                )
                gds.append(d)

            sds = []
            for i in range(2 * NBUF):
                gds[i].wait()
                sds.append(
                    pltpu.async_copy(
                        rows[i // NBUF][i % NBUF], acc.at[dst_v.at[base + i]],
                        ssem[i // NBUF][i % NBUF], add=True,
                    )
                )
            for d in sds:
                d.wait()
            return carry

        lax.fori_loop(0, NG2 - 1, super_body, 0)

        base = (NG2 - 1) * 2 * NBUF
        gds = [None] * (2 * NBUF)
        for i in range(2 * NBUF):
            def gissue(i=i):
                gds[i] = pltpu.async_copy(
                    feat_hbm.at[src_v.at[base + i]], rows[i // NBUF][i % NBUF],
                    gsem[i // NBUF][i % NBUF],
                )
            pl.when(base + i < cnt)(gissue)
        for i in range(2 * NBUF):
            def gwait(i=i):
                gds[i].wait()
                pltpu.sync_copy(
                    rows[i // NBUF][i % NBUF], acc.at[dst_v.at[base + i]], add=True
                )
            pl.when(base + i < cnt)(gwait)

        plsc.subcore_barrier()
        if split_out:
            for k in range(nsplit):
                pltpu.sync_copy(
                    acc.at[sl, pl.ds(k * L, L)], outs[k].at[c, sl]
                )
        else:
            pltpu.sync_copy(acc.at[sl], outs[0].at[c, sl])

    out_t = [jax.ShapeDtypeStruct((NC, NPAD, L if split_out else F), jnp.float32)
             for _ in range(nsplit)]
    return pl.kernel(
        body,
        out_type=out_t if split_out else out_t[0],
        mesh=_sc_mesh(),
        compiler_params=_SC_PARAMS,
        scratch_types=[
            pltpu.VMEM_SHARED((NPAD, F), jnp.float32),
            pltpu.VMEM((CPWB + 1, CHUNK), jnp.int32),
            pltpu.VMEM((CPWB + 1, CHUNK), jnp.int32),
            pltpu.VMEM((128, F), jnp.float32),
            [[pltpu.VMEM((CHUNK, F), jnp.float32) for _ in range(NBUF)] for _ in range(2)],
            [[pltpu.SemaphoreType.DMA for _ in range(NBUF)] for _ in range(2)],
            [[pltpu.SemaphoreType.DMA for _ in range(NBUF)] for _ in range(2)],
        ],
    )


def _make_degree():
    def body(edge_hbm, out_hbm, acc, dst_v, ones_v, zbuf):
        c = lax.axis_index("c")
        s = lax.axis_index("s")
        wid = s * NC + c
        sl = pl.ds(s * RPT, RPT)

        def zfill(i, carry):
            zbuf[i, :] = jnp.zeros((L,), jnp.float32)
            return carry

        lax.fori_loop(0, 128, zfill, 0)
        for k in range(RPT // 128):
            pltpu.sync_copy(zbuf, acc.at[pl.ds(s * RPT + k * 128, 128)])

        def ones_body(i, carry):
            ones_v[i, :] = jnp.ones((L,), jnp.float32)
            return carry

        lax.fori_loop(0, CHUNK, ones_body, 0)
        cnt = jnp.where(wid < NCHUNK - NW * CPWB, CPWB + 1, CPWB)
        start = CPWB * wid + jnp.minimum(wid, NCHUNK - NW * CPWB)
        pltpu.sync_copy(edge_hbm.at[1, pl.ds(start, CPWB)], dst_v.at[pl.ds(0, CPWB)])

        @pl.when(cnt > CPWB)
        def _():
            pltpu.sync_copy(edge_hbm.at[1, start + CPWB], dst_v.at[CPWB])

        plsc.subcore_barrier()

        def jbody(j, carry):
            pltpu.sync_copy(ones_v, acc.at[dst_v.at[j]], add=True)
            return carry

        lax.fori_loop(0, CPWB, jbody, 0)

        @pl.when(cnt > CPWB)
        def _():
            pltpu.sync_copy(ones_v, acc.at[dst_v.at[CPWB]], add=True)

        plsc.subcore_barrier()
        pltpu.sync_copy(acc.at[sl], out_hbm.at[c, sl])

    return pl.kernel(
        body,
        out_type=jax.ShapeDtypeStruct((NC, NPAD, L), jnp.float32),
        mesh=_sc_mesh(),
        compiler_params=_SC_PARAMS,
        scratch_types=[
            pltpu.VMEM_SHARED((NPAD, L), jnp.float32),
            pltpu.VMEM((CPWB + 1, CHUNK), jnp.int32),
            pltpu.VMEM((CHUNK, L), jnp.float32),
            pltpu.VMEM((128, L), jnp.float32),
        ],
    )


_propagate16 = _make_propagate(HID, split_out=False)
_propagate48 = _make_propagate(CLSP, split_out=True)
_degree = _make_degree()

_RB = 128
_GRID = NPAD // (8 * _RB)
_XR = N * F_IN // 1024
_R16 = NPAD * HID // 128
_W48 = 8 * CLSP


def _stage1(xv, bdw1, degv, m48):
    def body(xv_ref, w_ref, degv_ref, m48_ref, hs_ref, d16_ref, d48_ref):
        d16 = lax.rsqrt(degv_ref[0] + degv_ref[1] + 1.0)
        h = jnp.dot(xv_ref[...], w_ref[...], preferred_element_type=jnp.float32)
        hs_ref[...] = h * d16
        d16_ref[...] = d16
        d48_ref[...] = jnp.dot(d16, m48_ref[...], preferred_element_type=jnp.float32)

    return pl.pallas_call(
        body,
        grid=(_GRID,),
        in_specs=[
            pl.BlockSpec((_RB, 1024), lambda i: (i, 0)),
            pl.BlockSpec((1024, 128), lambda i: (0, 0)),
            pl.BlockSpec((NC, _RB, 128), lambda i: (0, i, 0)),
            pl.BlockSpec((128, _W48), lambda i: (0, 0)),
        ],
        out_specs=[
            pl.BlockSpec((_RB, 128), lambda i: (i, 0)),
            pl.BlockSpec((_RB, 128), lambda i: (i, 0)),
            pl.BlockSpec((_RB, _W48), lambda i: (i, 0)),
        ],
        out_shape=[
            jax.ShapeDtypeStruct((_R16, 128), jnp.float32),
            jax.ShapeDtypeStruct((_R16, 128), jnp.float32),
            jax.ShapeDtypeStruct((_R16, _W48), jnp.float32),
        ],
    )(xv, bdw1, degv, m48)


def _stage2(p1v, hs_pk, d16, d48, b1t, bdw2):
    def body(p_ref, hs_ref, d16_ref, d48_ref, b1_ref, w_ref, hs2_ref):
        sacc = d16_ref[...] * (p_ref[0] + p_ref[1] + hs_ref[...]) + b1_ref[...]
        h1 = jnp.maximum(sacc, 0.0)
        t = jnp.dot(h1, w_ref[...], preferred_element_type=jnp.float32)
        hs2_ref[...] = t * d48_ref[...]

    return pl.pallas_call(
        body,
        grid=(_GRID,),
        in_specs=[
            pl.BlockSpec((NC, _RB, 128), lambda i: (0, i, 0)),
            pl.BlockSpec((_RB, 128), lambda i: (i, 0)),
            pl.BlockSpec((_RB, 128), lambda i: (i, 0)),
            pl.BlockSpec((_RB, _W48), lambda i: (i, 0)),
            pl.BlockSpec((1, 128), lambda i: (0, 0)),
            pl.BlockSpec((128, _W48), lambda i: (0, 0)),
        ],
        out_specs=pl.BlockSpec((_RB, _W48), lambda i: (i, 0)),
        out_shape=jax.ShapeDtypeStruct((_R16, _W48), jnp.float32),
    )(p1v, hs_pk, d16, d48, b1t, bdw2)


def _stage3(p2cg, hs2_pk, d48, b2t, seg, perm):
    def body(pa_ref, pb_ref, pc_ref, hs2_ref, d48_ref, b2_ref, seg_ref, perm_ref,
             out_ref, logit_ref):
        p_cg = jnp.concatenate(
            [r[0] + r[1] for r in (pa_ref, pb_ref, pc_ref)], axis=1
        )
        p_pk = jnp.dot(p_cg, perm_ref[...], preferred_element_type=jnp.float32)
        tr = d48_ref[...] * (p_pk + hs2_ref[...]) + b2_ref[...]
        m = jnp.max(tr, axis=1, keepdims=True)
        e = jnp.exp(tr - m)
        ssum = jnp.dot(e, seg_ref[...], preferred_element_type=jnp.float32)
        out_ref[...] = e / ssum
        logit_ref[...] = tr

    cg_spec = pl.BlockSpec((NC, _RB, 128), lambda i: (0, i, 0))
    return pl.pallas_call(
        body,
        grid=(_GRID,),
        in_specs=[
            cg_spec,
            cg_spec,
            cg_spec,
            pl.BlockSpec((_RB, _W48), lambda i: (i, 0)),
            pl.BlockSpec((_RB, _W48), lambda i: (i, 0)),
            pl.BlockSpec((1, _W48), lambda i: (0, 0)),
            pl.BlockSpec((_W48, _W48), lambda i: (0, 0)),
            pl.BlockSpec((_W48, _W48), lambda i: (0, 0)),
        ],
        out_specs=[
            pl.BlockSpec((_RB, _W48), lambda i: (i, 0)),
            pl.BlockSpec((_RB, _W48), lambda i: (i, 0)),
        ],
        out_shape=[
            jax.ShapeDtypeStruct((_R16, _W48), jnp.float32),
            jax.ShapeDtypeStruct((_R16, _W48), jnp.float32),
        ],
    )(*p2cg, hs2_pk, d48, b2t, seg, perm)


def kernel(x, edge_index, W1, b1, W2, b2):
    ev = edge_index.reshape(2, NCHUNK, CHUNK)
    xv = x.reshape(_XR, 1024)
    eye8 = jnp.eye(8, dtype=jnp.float32)
    bdw1 = jnp.kron(eye8, W1)
    w2p = jnp.zeros((HID, CLSP), jnp.float32).at[:, :CLS].set(W2)
    bdw2 = jnp.kron(eye8, w2p)
    pick = jnp.zeros((HID, CLSP), jnp.float32).at[0, :].set(1.0)
    m48 = jnp.kron(eye8, pick)
    b1t = jnp.tile(b1, 8).reshape(1, 128)
    b2p = jnp.full((CLSP,), -1e30, jnp.float32).at[:CLS].set(b2)
    b2t = jnp.tile(b2p, 8).reshape(1, _W48)
    seg = jnp.kron(eye8, jnp.ones((CLSP, CLSP), jnp.float32))
    ii = jnp.arange(_W48)
    jj = 48 * ((ii % 128) // 16) + 16 * (ii // 128) + (ii % 16)
    perm = jnp.zeros((_W48, _W48), jnp.float32).at[ii, jj].set(1.0)

    degp = _degree(ev)
    degv = degp.reshape(NC, _R16, 128)
    hs_pk, d16, d48 = _stage1(xv, bdw1, degv, m48)

    p1 = _propagate16(hs_pk.reshape(NPAD, HID), ev)
    p1v = p1.reshape(NC, _R16, 128)
    hs2_pk = _stage2(p1v, hs_pk, d16, d48, b1t, bdw2)

    p2 = _propagate48(hs2_pk.reshape(NPAD, CLSP), ev)
    p2cg = [pk.reshape(NC, _R16, 128) for pk in p2]
    out_pk, logit_pk = _stage3(p2cg, hs2_pk, d48, b2t, seg, perm)
    out = out_pk.reshape(NPAD, CLSP)[:N, :CLS]
    logits = logit_pk.reshape(NPAD, CLSP)[:N, :CLS]
    return (out, logits)


# speedup vs baseline: 55.7070x; 1.0552x over previous
import jax
import jax.numpy as jnp
from jax import lax
from jax.experimental import pallas as pl
from jax.experimental.pallas import tpu as pltpu
from jax.experimental.pallas import tpu_sc as plsc

N = 10000
E = 320000
F_IN = 128
HID = 16
CLS = 40
CLSP = 48

NC, NS, L = 2, 16, 16
NW = NC * NS
CHUNK = 128
NCHUNK = E // CHUNK
CPWB = NCHUNK // NW
NPAD = 10240
RPT = NPAD // NS


def _sc_mesh():
    return plsc.VectorSubcoreMesh(
        core_axis_name="c", subcore_axis_name="s", num_cores=NC, num_subcores=NS
    )


_SC_PARAMS = pltpu.CompilerParams(use_tc_tiling_on_sc=False)


def _make_propagate(F, nbuf):
    ng = (CPWB + 1 + 2 * nbuf - 1) // (2 * nbuf)

    def body(feat_hbm, edge_hbm, out_hbm, acc, src_v, dst_v, zbuf, rows, gsem, ssem):
        c = lax.axis_index("c")
        s = lax.axis_index("s")
        wid = s * NC + c
        cnt = jnp.where(wid < NCHUNK - NW * CPWB, CPWB + 1, CPWB)
        start = CPWB * wid + jnp.minimum(wid, NCHUNK - NW * CPWB)
        sl = pl.ds(s * RPT, RPT)

        def zfill(i, carry):
            for k in range(F // L):
                zbuf[i, pl.ds(k * L, L)] = jnp.zeros((L,), jnp.float32)
            return carry

        lax.fori_loop(0, 128, zfill, 0)
        for k in range(RPT // 128):
            pltpu.sync_copy(zbuf, acc.at[pl.ds(s * RPT + k * 128, 128)])
        pltpu.sync_copy(edge_hbm.at[0, pl.ds(start, CPWB)], src_v.at[pl.ds(0, CPWB)])
        pltpu.sync_copy(edge_hbm.at[1, pl.ds(start, CPWB)], dst_v.at[pl.ds(0, CPWB)])

        @pl.when(cnt > CPWB)
        def _():
            pltpu.sync_copy(edge_hbm.at[0, start + CPWB], src_v.at[CPWB])
            pltpu.sync_copy(edge_hbm.at[1, start + CPWB], dst_v.at[CPWB])

        plsc.subcore_barrier()

        def super_body(u, carry):
            base = u * 2 * nbuf
            gds = []
            for i in range(2 * nbuf):
                d = pltpu.async_copy(
                    feat_hbm.at[src_v.at[base + i]], rows[i // nbuf][i % nbuf],
                    gsem[i // nbuf][i % nbuf],
                )
                gds.append(d)

            sds = []
            for i in range(2 * nbuf):
                gds[i].wait()
                sds.append(
                    pltpu.async_copy(
                        rows[i // nbuf][i % nbuf], acc.at[dst_v.at[base + i]],
                        ssem[i // nbuf][i % nbuf], add=True,
                    )
                )
            for d in sds:
                d.wait()
            return carry

        lax.fori_loop(0, ng - 1, super_body, 0)

        base = (ng - 1) * 2 * nbuf
        gds = [None] * (2 * nbuf)
        for i in range(2 * nbuf):
            def gissue(i=i):
                gds[i] = pltpu.async_copy(
                    feat_hbm.at[src_v.at[base + i]], rows[i // nbuf][i % nbuf],
                    gsem[i // nbuf][i % nbuf],
                )
            pl.when(base + i < cnt)(gissue)
        for i in range(2 * nbuf):
            def gwait(i=i):
                gds[i].wait()
                pltpu.sync_copy(
                    rows[i // nbuf][i % nbuf], acc.at[dst_v.at[base + i]], add=True
                )
            pl.when(base + i < cnt)(gwait)

        plsc.subcore_barrier()
        pltpu.sync_copy(acc.at[sl], out_hbm.at[c, sl])

    return pl.kernel(
        body,
        out_type=jax.ShapeDtypeStruct((NC, NPAD, F), jnp.float32),
        mesh=_sc_mesh(),
        compiler_params=_SC_PARAMS,
        scratch_types=[
            pltpu.VMEM_SHARED((NPAD, F), jnp.float32),
            pltpu.VMEM((ng * 2 * nbuf, CHUNK), jnp.int32),
            pltpu.VMEM((ng * 2 * nbuf, CHUNK), jnp.int32),
            pltpu.VMEM((128, F), jnp.float32),
            [[pltpu.VMEM((CHUNK, F), jnp.float32) for _ in range(nbuf)] for _ in range(2)],
            [[pltpu.SemaphoreType.DMA for _ in range(nbuf)] for _ in range(2)],
            [[pltpu.SemaphoreType.DMA for _ in range(nbuf)] for _ in range(2)],
        ],
    )


def _make_degree():
    def body(edge_hbm, out_hbm, acc, dst_v, ones_v, zbuf):
        c = lax.axis_index("c")
        s = lax.axis_index("s")
        wid = s * NC + c
        sl = pl.ds(s * RPT, RPT)

        def zfill(i, carry):
            zbuf[i, :] = jnp.zeros((L,), jnp.float32)
            return carry

        lax.fori_loop(0, 128, zfill, 0)
        for k in range(RPT // 128):
            pltpu.sync_copy(zbuf, acc.at[pl.ds(s * RPT + k * 128, 128)])

        def ones_body(i, carry):
            ones_v[i, :] = jnp.ones((L,), jnp.float32)
            return carry

        lax.fori_loop(0, CHUNK, ones_body, 0)
        cnt = jnp.where(wid < NCHUNK - NW * CPWB, CPWB + 1, CPWB)
        start = CPWB * wid + jnp.minimum(wid, NCHUNK - NW * CPWB)
        pltpu.sync_copy(edge_hbm.at[1, pl.ds(start, CPWB)], dst_v.at[pl.ds(0, CPWB)])

        @pl.when(cnt > CPWB)
        def _():
            pltpu.sync_copy(edge_hbm.at[1, start + CPWB], dst_v.at[CPWB])

        plsc.subcore_barrier()

        def jbody(j, carry):
            pltpu.sync_copy(ones_v, acc.at[dst_v.at[j]], add=True)
            return carry

        lax.fori_loop(0, CPWB, jbody, 0)

        @pl.when(cnt > CPWB)
        def _():
            pltpu.sync_copy(ones_v, acc.at[dst_v.at[CPWB]], add=True)

        plsc.subcore_barrier()
        pltpu.sync_copy(acc.at[sl], out_hbm.at[c, sl])

    return pl.kernel(
        body,
        out_type=jax.ShapeDtypeStruct((NC, NPAD, L), jnp.float32),
        mesh=_sc_mesh(),
        compiler_params=_SC_PARAMS,
        scratch_types=[
            pltpu.VMEM_SHARED((NPAD, L), jnp.float32),
            pltpu.VMEM((80, CHUNK), jnp.int32),
            pltpu.VMEM((CHUNK, L), jnp.float32),
            pltpu.VMEM((128, L), jnp.float32),
        ],
    )


_propagate16 = _make_propagate(HID, nbuf=4)
_propagate48 = _make_propagate(CLSP, nbuf=4)
_degree = _make_degree()

_RB = 128
_GRID = NPAD // (8 * _RB)
_XR = N * F_IN // 1024
_R16 = NPAD * HID // 128
_W48 = 8 * CLSP


def _stage1(xv, bdw1, degv, m48):
    def body(xv_ref, w_ref, degv_ref, m48_ref, hs_ref, d16_ref, d48_ref):
        d16 = lax.rsqrt(degv_ref[0] + degv_ref[1] + 1.0)
        h = jnp.dot(xv_ref[...], w_ref[...], preferred_element_type=jnp.float32)
        hs_ref[...] = h * d16
        d16_ref[...] = d16
        d48_ref[...] = jnp.dot(d16, m48_ref[...], preferred_element_type=jnp.float32)

    return pl.pallas_call(
        body,
        grid=(_GRID,),
        in_specs=[
            pl.BlockSpec((_RB, 1024), lambda i: (i, 0)),
            pl.BlockSpec((1024, 128), lambda i: (0, 0)),
            pl.BlockSpec((NC, _RB, 128), lambda i: (0, i, 0)),
            pl.BlockSpec((128, _W48), lambda i: (0, 0)),
        ],
        out_specs=[
            pl.BlockSpec((_RB, 128), lambda i: (i, 0)),
            pl.BlockSpec((_RB, 128), lambda i: (i, 0)),
            pl.BlockSpec((_RB, _W48), lambda i: (i, 0)),
        ],
        out_shape=[
            jax.ShapeDtypeStruct((_R16, 128), jnp.float32),
            jax.ShapeDtypeStruct((_R16, 128), jnp.float32),
            jax.ShapeDtypeStruct((_R16, _W48), jnp.float32),
        ],
    )(xv, bdw1, degv, m48)


def _stage2(p1v, hs_pk, d16, d48, b1t, bdw2):
    def body(p_ref, hs_ref, d16_ref, d48_ref, b1_ref, w_ref, hs2_ref):
        sacc = d16_ref[...] * (p_ref[0] + p_ref[1] + hs_ref[...]) + b1_ref[...]
        h1 = jnp.maximum(sacc, 0.0)
        t = jnp.dot(h1, w_ref[...], preferred_element_type=jnp.float32)
        hs2_ref[...] = t * d48_ref[...]

    return pl.pallas_call(
        body,
        grid=(_GRID,),
        in_specs=[
            pl.BlockSpec((NC, _RB, 128), lambda i: (0, i, 0)),
            pl.BlockSpec((_RB, 128), lambda i: (i, 0)),
            pl.BlockSpec((_RB, 128), lambda i: (i, 0)),
            pl.BlockSpec((_RB, _W48), lambda i: (i, 0)),
            pl.BlockSpec((1, 128), lambda i: (0, 0)),
            pl.BlockSpec((128, _W48), lambda i: (0, 0)),
        ],
        out_specs=pl.BlockSpec((_RB, _W48), lambda i: (i, 0)),
        out_shape=jax.ShapeDtypeStruct((_R16, _W48), jnp.float32),
    )(p1v, hs_pk, d16, d48, b1t, bdw2)


def _stage3(p2v, hs2_pk, d48, b2t, seg, s012):
    def body(p_ref, hs2_ref, d48_ref, b2_ref, seg_ref, s_ref, *outs):
        x = p_ref[0] + p_ref[1]
        y = jnp.dot(s_ref[...], x, preferred_element_type=jnp.float32)
        p_pk = jnp.concatenate([y[0:128], y[128:256], y[256:384]], axis=1)
        tr = d48_ref[...] * (p_pk + hs2_ref[...]) + b2_ref[...]
        m = jnp.max(tr, axis=1, keepdims=True)
        e = jnp.exp(tr - m)
        ssum = jnp.dot(e, seg_ref[...], preferred_element_type=jnp.float32)
        outs[0][...] = e / ssum
        outs[1][...] = tr

    o_spec = pl.BlockSpec((_RB, _W48), lambda i: (i, 0))
    return pl.pallas_call(
        body,
        grid=(_GRID,),
        in_specs=[
            pl.BlockSpec((NC, 3 * _RB, 128), lambda i: (0, i, 0)),
            pl.BlockSpec((_RB, _W48), lambda i: (i, 0)),
            pl.BlockSpec((_RB, _W48), lambda i: (i, 0)),
            pl.BlockSpec((1, _W48), lambda i: (0, 0)),
            pl.BlockSpec((_W48, _W48), lambda i: (0, 0)),
            pl.BlockSpec((_W48, _W48), lambda i: (0, 0)),
        ],
        out_specs=[o_spec] * 2,
        out_shape=[jax.ShapeDtypeStruct((_R16, _W48), jnp.float32)] * 2,
    )(p2v, hs2_pk, d48, b2t, seg, s012)


def kernel(x, edge_index, W1, b1, W2, b2):
    ev = edge_index.reshape(2, NCHUNK, CHUNK)
    xv = x.reshape(_XR, 1024)
    eye8 = jnp.eye(8, dtype=jnp.float32)
    bdw1 = jnp.kron(eye8, W1)
    w2p = jnp.zeros((HID, CLSP), jnp.float32).at[:, :CLS].set(W2)
    bdw2 = jnp.kron(eye8, w2p)
    pick = jnp.zeros((HID, CLSP), jnp.float32).at[0, :].set(1.0)
    m48 = jnp.kron(eye8, pick)
    b1t = jnp.tile(b1, 8).reshape(1, 128)
    b2p = jnp.full((CLSP,), -1e30, jnp.float32).at[:CLS].set(b2)
    b2t = jnp.tile(b2p, 8).reshape(1, _W48)
    seg = jnp.kron(eye8, jnp.ones((CLSP, CLSP), jnp.float32))
    ii = jnp.arange(_W48)
    s012 = (ii[None, :] == (3 * (ii % 128) + ii // 128)[:, None]).astype(jnp.float32)

    degp = _degree(ev)
    degv = degp.reshape(NC, _R16, 128)
    hs_pk, d16, d48 = _stage1(xv, bdw1, degv, m48)

    p1 = _propagate16(hs_pk.reshape(NPAD, HID), ev)
    p1v = p1.reshape(NC, _R16, 128)
    hs2_pk = _stage2(p1v, hs_pk, d16, d48, b1t, bdw2)

    p2 = _propagate48(hs2_pk.reshape(NPAD, CLSP), ev)
    p2v = p2.reshape(NC, 3 * _R16, 128)
    out_pk, logit_pk = _stage3(p2v, hs2_pk, d48, b2t, seg, s012)
    out = out_pk.reshape(NPAD, CLSP)[:N, :CLS]
    logits = logit_pk.reshape(NPAD, CLSP)[:N, :CLS]
    return (out, logits)
